# baseline (speedup 1.0000x reference)
import functools
import math

import jax
import jax.numpy as jnp
from jax import lax
from jax.experimental import pallas as pl
from jax.experimental.pallas import tpu as pltpu
from jax.experimental.pallas import tpu_sc as plsc

_C = 16
_K = 16
_DEPTH = 4
_M = 64
_R = 2048
_CH = 256
_NW = 32


def _encode_body(i_ref, w_ref, t_ref, o_ref):
    x = jnp.dot(i_ref[...].astype(jnp.bfloat16), w_ref[...],
                preferred_element_type=jnp.float32)
    x = x.astype(jnp.bfloat16).astype(jnp.float32)
    v0 = x[:, 0:16]
    v1 = x[:, 16:32]
    v2 = x[:, 32:48]
    v3 = x[:, 48:64]

    def tr(j):
        return t_ref[j, :][None, :]

    b0 = v0 > tr(0)
    t1 = jnp.where(b0, tr(2), tr(1))
    b1 = v1 > t1
    t2 = jnp.where(b0, jnp.where(b1, tr(6), tr(5)),
                   jnp.where(b1, tr(4), tr(3)))
    b2 = v2 > t2
    t3 = jnp.where(
        b0,
        jnp.where(b1, jnp.where(b2, tr(14), tr(13)),
                  jnp.where(b2, tr(12), tr(11))),
        jnp.where(b1, jnp.where(b2, tr(10), tr(9)),
                  jnp.where(b2, tr(8), tr(7))))
    b3 = v3 > t3
    k = (b0.astype(jnp.int32) * 8 + b1.astype(jnp.int32) * 4
         + b2.astype(jnp.int32) * 2 + b3.astype(jnp.int32))
    c_iota = lax.broadcasted_iota(jnp.int32, k.shape, 1)
    o_ref[...] = c_iota * (_K * _M) + k * _M


def _tc_encode(I, W, Trp):
    n = I.shape[0]
    d = I.shape[1]
    grid = n // _R
    return pl.pallas_call(
        _encode_body,
        grid=(grid,),
        in_specs=[
            pl.BlockSpec((_R, d), lambda i: (i, 0)),
            pl.BlockSpec((d, 64), lambda i: (0, 0)),

            pl.BlockSpec((16, 16), lambda i: (0, 0)),
        ],
        out_specs=pl.BlockSpec((_R, _C), lambda i: (i, 0)),
        out_shape=jax.ShapeDtypeStruct((n, _C), jnp.int32),
    )(I, W, Trp)


def _sc_gather_fn(n):
    rows_per_w = n // _NW
    nch = rows_per_w // _CH
    mesh = plsc.VectorSubcoreMesh(core_axis_name="c", subcore_axis_name="s")

    @functools.partial(
        pl.kernel,
        mesh=mesh,
        out_type=jax.ShapeDtypeStruct((n * _M,), jnp.float32),
        scratch_types=[
            pltpu.VMEM((_C * _K * _M,), jnp.float32),
            pltpu.VMEM((_CH * _C,), jnp.int32),
            pltpu.VMEM((_CH * _M,), jnp.float32),
        ],
    )
    def sc_gather(off_hbm, g_hbm, out_hbm, g_v, idx_v, out_v):
        wid = lax.axis_index("s") * 2 + lax.axis_index("c")
        pltpu.sync_copy(g_hbm, g_v)
        base = wid * rows_per_w

        def chunk_body(ci, _):
            row0 = base + ci * _CH
            pltpu.sync_copy(off_hbm.at[pl.ds(row0 * _C, _CH * _C)], idx_v)

            def row_body(r, _):
                rb = r * _C
                ov = idx_v[pl.ds(rb, _C)]
                a0 = jnp.zeros((16,), jnp.float32)
                a1 = jnp.zeros((16,), jnp.float32)
                a2 = jnp.zeros((16,), jnp.float32)
                a3 = jnp.zeros((16,), jnp.float32)
                for c in range(_C):
                    o = ov[c]
                    a0 = a0 + g_v[pl.ds(o, 16)]
                    a1 = a1 + g_v[pl.ds(o + 16, 16)]
                    a2 = a2 + g_v[pl.ds(o + 32, 16)]
                    a3 = a3 + g_v[pl.ds(o + 48, 16)]
                ob = r * _M
                out_v[pl.ds(ob, 16)] = a0
                out_v[pl.ds(ob + 16, 16)] = a1
                out_v[pl.ds(ob + 32, 16)] = a2
                out_v[pl.ds(ob + 48, 16)] = a3
                return 0

            lax.fori_loop(0, _CH, row_body, 0)
            pltpu.sync_copy(out_v, out_hbm.at[pl.ds(row0 * _M, _CH * _M)])
            return 0

        lax.fori_loop(0, nch, chunk_body, 0)

    return sc_gather


def kernel(I, T, L, S, B, A):
    n, d = I.shape
    W = (A[:, :, :, None] * jnp.eye(_C, dtype=A.dtype)[:, None, None, :])
    W = W.reshape(d, _DEPTH * _C).astype(jnp.bfloat16)
    Tr = T.reshape(_C, _K - 1).T
    Trp = jnp.concatenate([Tr, jnp.zeros((1, _C), Tr.dtype)], axis=0)
    G = jnp.transpose(L, (1, 2, 0)).reshape(-1)

    off = _tc_encode(I, W, Trp)
    out = _sc_gather_fn(n)(off.reshape(-1), G)
    return out.reshape(n, _M)


# speedup vs baseline: 1.8730x; 1.8730x over previous
import functools
import math

import jax
import jax.numpy as jnp
from jax import lax
from jax.experimental import pallas as pl
from jax.experimental.pallas import tpu as pltpu
from jax.experimental.pallas import tpu_sc as plsc

_C = 16
_K = 16
_NP = 7
_DEPTH = 4
_M = 64
_R = 4096
_CH = 128
_NW = 32
_PAIR_WORDS = _NP * _K * _K * _M
_S7 = _PAIR_WORDS
_S15 = _PAIR_WORDS + _K * _M
_GWORDS = _PAIR_WORDS + 2 * _K * _M


def _encode_body(i_ref, w_ref, t_ref, o_ref):
    xt = lax.dot_general(w_ref[...], i_ref[...].astype(jnp.bfloat16),
                         (((1,), (1,)), ((), ())),
                         preferred_element_type=jnp.float32)
    x = xt.astype(jnp.bfloat16).astype(jnp.float32)
    v0 = x[0:16, :]
    v1 = x[16:32, :]
    v2 = x[32:48, :]
    v3 = x[48:64, :]

    def tr(j):
        return t_ref[:, j][:, None]

    b0 = v0 > tr(0)
    t1 = jnp.where(b0, tr(2), tr(1))
    b1 = v1 > t1
    t2 = jnp.where(b0, jnp.where(b1, tr(6), tr(5)),
                   jnp.where(b1, tr(4), tr(3)))
    b2 = v2 > t2
    t3 = jnp.where(
        b0,
        jnp.where(b1, jnp.where(b2, tr(14), tr(13)),
                  jnp.where(b2, tr(12), tr(11))),
        jnp.where(b1, jnp.where(b2, tr(10), tr(9)),
                  jnp.where(b2, tr(8), tr(7))))
    b3 = v3 > t3
    k = (b0.astype(jnp.int32) * 8 + b1.astype(jnp.int32) * 4
         + b2.astype(jnp.int32) * 2 + b3.astype(jnp.int32))
    k1 = k[0:8, :]
    k2 = k[8:16, :]
    p_iota = lax.broadcasted_iota(jnp.int32, k1.shape, 0)
    base = jnp.where(p_iota < _NP, p_iota * (_K * _K * _M), 0)
    o_ref[...] = base + k1 * (_K * _M) + k2 * _M


def _tc_encode(I, WT, Trp):
    n, d = I.shape
    grid = n // _R
    return pl.pallas_call(
        _encode_body,
        grid=(grid,),
        in_specs=[
            pl.BlockSpec((_R, d), lambda i: (i, 0)),
            pl.BlockSpec((64, d), lambda i: (0, 0)),
            pl.BlockSpec((16, 16), lambda i: (0, 0)),
        ],
        out_specs=pl.BlockSpec((8, _R), lambda i: (0, i)),
        out_shape=jax.ShapeDtypeStruct((8, n), jnp.int32),
    )(I, WT, Trp)


def _sc_gather_fn(n):
    rows_per_w = n // _NW
    nch = rows_per_w // _CH
    mesh = plsc.VectorSubcoreMesh(core_axis_name="c", subcore_axis_name="s")

    @functools.partial(
        pl.kernel,
        mesh=mesh,
        out_type=jax.ShapeDtypeStruct((n * _M,), jnp.float32),
        scratch_types=[
            pltpu.VMEM((_GWORDS,), jnp.float32),
            pltpu.VMEM((8, _CH), jnp.int32),
            pltpu.VMEM((_CH * _M,), jnp.float32),
        ],
    )
    def sc_gather(off_hbm, g_hbm, out_hbm, g_v, idx_v, out_v):
        wid = lax.axis_index("s") * 2 + lax.axis_index("c")
        pltpu.sync_copy(g_hbm, g_v)
        base = wid * rows_per_w

        def chunk_body(ci, _):
            row0 = base + ci * _CH
            pltpu.sync_copy(off_hbm.at[:, pl.ds(row0, _CH)], idx_v)

            def grp_body(gi, _):
                g16 = gi * 16
                ovs = [idx_v[p, pl.ds(g16, 16)] for p in range(8)]
                for r16 in range(16):
                    accs = None
                    for p in range(_NP):
                        o = pl.multiple_of(ovs[p][r16], 64)
                        ld = [g_v[pl.ds(o + 16 * g, 16)] for g in range(4)]
                        if accs is None:
                            accs = ld
                        else:
                            accs = [a + b for a, b in zip(accs, ld)]
                    vv = ovs[7][r16]
                    o7 = pl.multiple_of(_S7 + ((vv >> 4) & 0x3C0), 64)
                    o15 = pl.multiple_of(_S15 + (vv & 0x3C0), 64)
                    l7 = [g_v[pl.ds(o7 + 16 * g, 16)] for g in range(4)]
                    l15 = [g_v[pl.ds(o15 + 16 * g, 16)] for g in range(4)]
                    accs = [a + (b + c) for a, b, c in zip(accs, l7, l15)]
                    ob = (g16 + r16) * _M
                    for g in range(4):
                        out_v[pl.ds(ob + 16 * g, 16)] = accs[g]
                return 0

            lax.fori_loop(0, _CH // 16, grp_body, 0)
            pltpu.sync_copy(out_v, out_hbm.at[pl.ds(row0 * _M, _CH * _M)])
            return 0

        lax.fori_loop(0, nch, chunk_body, 0)

    return sc_gather


def kernel(I, T, L, S, B, A):
    n, d = I.shape
    W = (A[:, :, :, None] * jnp.eye(_C, dtype=A.dtype)[:, None, None, :])
    WT = W.reshape(d, _DEPTH * _C).T.astype(jnp.bfloat16)
    Trp = jnp.concatenate(
        [T.reshape(_C, _K - 1), jnp.zeros((_C, 1), T.dtype)], axis=1)
    Lt = jnp.transpose(L, (1, 2, 0))
    G2 = (Lt[:_NP, :, None, :] + Lt[8:8 + _NP, None, :, :]).reshape(-1)
    G = jnp.concatenate([G2, Lt[7].reshape(-1), Lt[15].reshape(-1)])

    off = _tc_encode(I, WT, Trp)
    out = _sc_gather_fn(n)(off, G)
    return out.reshape(n, _M)


# speedup vs baseline: 2.6856x; 1.4338x over previous
import functools
import math

import jax
import jax.numpy as jnp
from jax import lax
from jax.experimental import pallas as pl
from jax.experimental.pallas import tpu as pltpu
from jax.experimental.pallas import tpu_sc as plsc

_C = 16
_K = 16
_NP = 6
_DEPTH = 4
_M = 64
_R = 4096
_CH = 128
_NW = 32
_PAIR_WORDS = _NP * _K * _K * _M
_SB = [_PAIR_WORDS + i * _K * _M for i in range(4)]
_GWORDS = _PAIR_WORDS + 4 * _K * _M


def _encode_body(i_ref, w_ref, t_ref, o_ref):
    xt = lax.dot_general(w_ref[...], i_ref[...].astype(jnp.bfloat16),
                         (((1,), (0,)), ((), ())),
                         preferred_element_type=jnp.float32)
    x = xt.astype(jnp.bfloat16).astype(jnp.float32)
    v0 = x[0:16, :]
    v1 = x[16:32, :]
    v2 = x[32:48, :]
    v3 = x[48:64, :]

    def tr(j):
        return t_ref[:, j][:, None]

    b0 = v0 > tr(0)
    t1 = jnp.where(b0, tr(2), tr(1))
    b1 = v1 > t1
    t2 = jnp.where(b0, jnp.where(b1, tr(6), tr(5)),
                   jnp.where(b1, tr(4), tr(3)))
    b2 = v2 > t2
    t3 = jnp.where(
        b0,
        jnp.where(b1, jnp.where(b2, tr(14), tr(13)),
                  jnp.where(b2, tr(12), tr(11))),
        jnp.where(b1, jnp.where(b2, tr(10), tr(9)),
                  jnp.where(b2, tr(8), tr(7))))
    b3 = v3 > t3
    k = (b0.astype(jnp.int32) * 8 + b1.astype(jnp.int32) * 4
         + b2.astype(jnp.int32) * 2 + b3.astype(jnp.int32))
    k1 = k[0:8, :]
    k2 = k[8:16, :]
    p_iota = lax.broadcasted_iota(jnp.int32, k1.shape, 0)
    base = jnp.where(p_iota < _NP, p_iota * (_K * _K * _M), 0)
    o_ref[...] = base + k1 * (_K * _M) + k2 * _M


def _tc_encode(IT, WT, Trp):
    d, n = IT.shape
    grid = n // _R
    return pl.pallas_call(
        _encode_body,
        grid=(grid,),
        in_specs=[
            pl.BlockSpec((d, _R), lambda i: (0, i)),
            pl.BlockSpec((64, d), lambda i: (0, 0)),
            pl.BlockSpec((16, 16), lambda i: (0, 0)),
        ],
        out_specs=pl.BlockSpec((8, _R), lambda i: (0, i)),
        out_shape=jax.ShapeDtypeStruct((8, n), jnp.int32),
    )(IT, WT, Trp)


def _sc_gather_fn(n):
    rows_per_w = n // _NW
    nch = rows_per_w // _CH
    mesh = plsc.VectorSubcoreMesh(core_axis_name="c", subcore_axis_name="s")

    @functools.partial(
        pl.kernel,
        mesh=mesh,
        out_type=jax.ShapeDtypeStruct((n, _M), jnp.float32),
        scratch_types=[
            pltpu.VMEM((_GWORDS,), jnp.float32),
            pltpu.VMEM((8, _CH), jnp.int32),
            pltpu.VMEM((_CH, _M), jnp.float32),
        ],
    )
    def sc_gather(off_hbm, g_hbm, out_hbm, g_v, idx_v, out_v):
        wid = lax.axis_index("s") * 2 + lax.axis_index("c")
        pltpu.sync_copy(g_hbm, g_v)
        base = wid * rows_per_w

        def chunk_body(ci, _):
            row0 = base + ci * _CH
            pltpu.sync_copy(off_hbm.at[:, pl.ds(row0, _CH)], idx_v)

            def grp_body(gi, _):
                g16 = gi * 16
                ovs = [idx_v[p, pl.ds(g16, 16)] for p in range(8)]
                for r16 in range(16):
                    accs = None
                    for p in range(_NP):
                        o = pl.multiple_of(ovs[p][r16], 64)
                        ld = [g_v[pl.ds(o + 16 * g, 16)] for g in range(4)]
                        if accs is None:
                            accs = ld
                        else:
                            accs = [a + b for a, b in zip(accs, ld)]
                    for j, row in enumerate((6, 7)):
                        vv = ovs[row][r16]
                        oa = pl.multiple_of(_SB[2 * j] + ((vv >> 4) & 0x3C0), 64)
                        ob_ = pl.multiple_of(_SB[2 * j + 1] + (vv & 0x3C0), 64)
                        la = [g_v[pl.ds(oa + 16 * g, 16)] for g in range(4)]
                        lb = [g_v[pl.ds(ob_ + 16 * g, 16)] for g in range(4)]
                        accs = [a + (b + c) for a, b, c in zip(accs, la, lb)]
                    orow = g16 + r16
                    for g in range(4):
                        out_v[orow, pl.ds(16 * g, 16)] = accs[g]
                return 0

            lax.fori_loop(0, _CH // 16, grp_body, 0)
            pltpu.sync_copy(out_v, out_hbm.at[pl.ds(row0, _CH), :])
            return 0

        lax.fori_loop(0, nch, chunk_body, 0)

    return sc_gather


def kernel(I, T, L, S, B, A):
    n, d = I.shape
    W = (A[:, :, :, None] * jnp.eye(_C, dtype=A.dtype)[:, None, None, :])
    WT = W.reshape(d, _DEPTH * _C).T.astype(jnp.bfloat16)
    Trp = jnp.concatenate(
        [T.reshape(_C, _K - 1), jnp.zeros((_C, 1), T.dtype)], axis=1)
    Lt = jnp.transpose(L, (1, 2, 0))
    G2 = (Lt[:_NP, :, None, :] + Lt[8:8 + _NP, None, :, :]).reshape(-1)
    G = jnp.concatenate([G2, Lt[6].reshape(-1), Lt[14].reshape(-1),
                         Lt[7].reshape(-1), Lt[15].reshape(-1)])

    off = _tc_encode(I.T, WT, Trp)
    return _sc_gather_fn(n)(off, G)


# speedup vs baseline: 2.9459x; 1.0969x over previous
import functools
import math

import jax
import jax.numpy as jnp
from jax import lax
from jax.experimental import pallas as pl
from jax.experimental.pallas import tpu as pltpu
from jax.experimental.pallas import tpu_sc as plsc

_C = 16
_K = 16
_NP = 5
_DEPTH = 4
_M = 64
_R = 4096
_CH = 128
_NW = 32
_PAIR_WORDS = _NP * _K * _K * _M
_SB = [_PAIR_WORDS + i * _K * _M for i in range(6)]
_GWORDS = _PAIR_WORDS + 6 * _K * _M


def _encode_body(i_ref, w_ref, t_ref, o_ref):
    xt = lax.dot_general(w_ref[...], i_ref[...].astype(jnp.bfloat16),
                         (((1,), (0,)), ((), ())),
                         preferred_element_type=jnp.float32)
    x = xt.astype(jnp.bfloat16).astype(jnp.float32)
    v0 = x[0:16, :]
    v1 = x[16:32, :]
    v2 = x[32:48, :]
    v3 = x[48:64, :]

    def tr(j):
        return t_ref[:, j][:, None]

    b0 = v0 > tr(0)
    t1 = jnp.where(b0, tr(2), tr(1))
    b1 = v1 > t1
    t2 = jnp.where(b0, jnp.where(b1, tr(6), tr(5)),
                   jnp.where(b1, tr(4), tr(3)))
    b2 = v2 > t2
    t3 = jnp.where(
        b0,
        jnp.where(b1, jnp.where(b2, tr(14), tr(13)),
                  jnp.where(b2, tr(12), tr(11))),
        jnp.where(b1, jnp.where(b2, tr(10), tr(9)),
                  jnp.where(b2, tr(8), tr(7))))
    b3 = v3 > t3
    k = (b0.astype(jnp.int32) * 8 + b1.astype(jnp.int32) * 4
         + b2.astype(jnp.int32) * 2 + b3.astype(jnp.int32))
    k1 = k[0:8, :]
    k2 = k[8:16, :]
    p_iota = lax.broadcasted_iota(jnp.int32, k1.shape, 0)
    base = jnp.where(p_iota < _NP, p_iota * (_K * _K * _M), 0)
    o_ref[...] = base + k1 * (_K * _M) + k2 * _M


def _tc_encode(IT, WT, Trp):
    d, n = IT.shape
    grid = n // _R
    return pl.pallas_call(
        _encode_body,
        grid=(grid,),
        in_specs=[
            pl.BlockSpec((d, _R), lambda i: (0, i)),
            pl.BlockSpec((64, d), lambda i: (0, 0)),
            pl.BlockSpec((16, 16), lambda i: (0, 0)),
        ],
        out_specs=pl.BlockSpec((8, _R), lambda i: (0, i)),
        out_shape=jax.ShapeDtypeStruct((8, n), jnp.int32),
    )(IT, WT, Trp)


def _sc_gather_fn(n):
    rows_per_w = n // _NW
    nch = rows_per_w // _CH
    mesh = plsc.VectorSubcoreMesh(core_axis_name="c", subcore_axis_name="s")

    @functools.partial(
        pl.kernel,
        mesh=mesh,
        out_type=jax.ShapeDtypeStruct((n, _M), jnp.float32),
        scratch_types=[
            pltpu.VMEM((_GWORDS,), jnp.float32),
            pltpu.VMEM((8, _CH), jnp.int32),
            pltpu.VMEM((8, _CH), jnp.int32),
            pltpu.VMEM((_CH, _M), jnp.float32),
            pltpu.VMEM((_CH, _M), jnp.float32),
            pltpu.SemaphoreType.DMA,
            pltpu.SemaphoreType.DMA,
            pltpu.SemaphoreType.DMA,
            pltpu.SemaphoreType.DMA,
        ],
    )
    def sc_gather(off_hbm, g_hbm, out_hbm, g_v, idx0, idx1, outv0, outv1,
                  si0, si1, so0, so1):
        wid = lax.axis_index("s") * 2 + lax.axis_index("c")
        base = wid * rows_per_w
        idx_bufs = (idx0, idx1)
        out_bufs = (outv0, outv1)
        si = (si0, si1)
        so = (so0, so1)

        def idx_src(ch_i):
            return off_hbm.at[:, pl.ds(base + ch_i * _CH, _CH)]

        pltpu.async_copy(idx_src(0), idx_bufs[0], si[0])
        pltpu.async_copy(idx_src(1), idx_bufs[1], si[1])
        pltpu.sync_copy(g_hbm, g_v)

        def process_chunk(ch_i, idx_v, out_v):
            def grp_body(gi, _):
                g16 = gi * 16
                ovs = [idx_v[p, pl.ds(g16, 16)] for p in range(8)]
                for r16 in range(16):
                    accs = None
                    for p in range(_NP):
                        o = pl.multiple_of(ovs[p][r16], 64)
                        ld = [g_v[pl.ds(o + 16 * g, 16)] for g in range(4)]
                        if accs is None:
                            accs = ld
                        else:
                            accs = [a + b for a, b in zip(accs, ld)]
                    for j, row in enumerate((5, 6, 7)):
                        vv = ovs[row][r16]
                        oa = pl.multiple_of(_SB[2 * j] + ((vv >> 4) & 0x3C0), 64)
                        ob_ = pl.multiple_of(_SB[2 * j + 1] + (vv & 0x3C0), 64)
                        la = [g_v[pl.ds(oa + 16 * g, 16)] for g in range(4)]
                        lb = [g_v[pl.ds(ob_ + 16 * g, 16)] for g in range(4)]
                        accs = [a + (b + c) for a, b, c in zip(accs, la, lb)]
                    orow = g16 + r16
                    for g in range(4):
                        out_v[orow, pl.ds(16 * g, 16)] = accs[g]
                return 0

            lax.fori_loop(0, _CH // 16, grp_body, 0)

        def pair_body(ci2, _):
            for b in range(2):
                ch_i = ci2 * 2 + b
                idx_v = idx_bufs[b]
                out_v = out_bufs[b]
                pltpu.make_async_copy(idx_src(ch_i), idx_v, si[b]).wait()

                @pl.when(ci2 > 0)
                def _wait_out():
                    pltpu.make_async_copy(
                        out_v, out_hbm.at[pl.ds(0, _CH), :], so[b]).wait()

                process_chunk(ch_i, idx_v, out_v)
                pltpu.async_copy(
                    out_v, out_hbm.at[pl.ds(base + ch_i * _CH, _CH), :], so[b])

                @pl.when(ch_i + 2 < nch)
                def _prefetch():
                    pltpu.async_copy(idx_src(ch_i + 2), idx_v, si[b])
            return 0

        lax.fori_loop(0, nch // 2, pair_body, 0)
        for b in range(2):
            pltpu.make_async_copy(
                out_bufs[b], out_hbm.at[pl.ds(0, _CH), :], so[b]).wait()

    return sc_gather


def kernel(I, T, L, S, B, A):
    n, d = I.shape
    W = (A[:, :, :, None] * jnp.eye(_C, dtype=A.dtype)[:, None, None, :])
    WT = W.reshape(d, _DEPTH * _C).T.astype(jnp.bfloat16)
    Trp = jnp.concatenate(
        [T.reshape(_C, _K - 1), jnp.zeros((_C, 1), T.dtype)], axis=1)
    Lt = jnp.transpose(L, (1, 2, 0))
    G2 = (Lt[:_NP, :, None, :] + Lt[8:8 + _NP, None, :, :]).reshape(-1)
    G = jnp.concatenate([G2] + [Lt[c].reshape(-1)
                                for c in (5, 13, 6, 14, 7, 15)])

    off = _tc_encode(I.T, WT, Trp)
    return _sc_gather_fn(n)(off, G)
